# baseline (speedup 1.0000x reference)
import functools

import jax
import jax.numpy as jnp
from jax import lax
from jax.experimental import pallas as pl
from jax.experimental.pallas import tpu as pltpu
from jax.experimental.pallas import tpu_sc as plsc

NC = 2
NS = 16
K = 128


def _make_sc_agg(NP, W, CH, npass):
  SLAB = NP // NS
  NB = 4
  mesh = plsc.VectorSubcoreMesh(core_axis_name="c", subcore_axis_name="s")

  @functools.partial(
      pl.kernel,
      out_type=jax.ShapeDtypeStruct((npass, NC, NP, W), jnp.float32),
      mesh=mesh,
      compiler_params=pltpu.CompilerParams(use_tc_tiling_on_sc=False),
      scratch_types=(
          [pltpu.VMEM((CH, K), jnp.int32)] * 2
          + [pltpu.VMEM((K, W), jnp.float32)] * NB
          + [pltpu.VMEM_SHARED((NP, W), jnp.float32)]
          + [pltpu.SemaphoreType.DMA] * (2 * NB)
      ),
  )
  def agg(*refs):
    rows_list = refs[:npass]
    src_hbm, dst_hbm, z_hbm, out_hbm = refs[npass:npass + 4]
    sidx, didx = refs[npass + 4:npass + 6]
    bufs = refs[npass + 6:npass + 6 + NB]
    acc = refs[npass + 6 + NB]
    sem_g = refs[npass + 7 + NB:npass + 7 + 2 * NB]
    sem_s = refs[npass + 7 + 2 * NB:]
    c = lax.axis_index("c")
    s = lax.axis_index("s")
    w = s * NC + c
    pltpu.sync_copy(src_hbm.at[pl.ds(w * CH, CH)], sidx)
    pltpu.sync_copy(dst_hbm.at[pl.ds(w * CH, CH)], didx)

    for p in range(npass):
      rows_hbm = rows_list[p]
      pltpu.sync_copy(z_hbm, acc.at[pl.ds(s * SLAB, SLAB)])
      plsc.subcore_barrier()

      for b in range(NB):
        pltpu.async_copy(rows_hbm.at[sidx.at[b]], bufs[b], sem_g[b])

      def body(i, carry):
        base = NB * i
        for b in range(NB):
          j = base + b
          pltpu.make_async_copy(rows_hbm.at[sidx.at[j]], bufs[b],
                                sem_g[b]).wait()
          pltpu.async_copy(bufs[b], acc.at[didx.at[j]], sem_s[b], add=True)
        for b in range(NB):
          j = base + b
          pltpu.make_async_copy(bufs[b], acc.at[didx.at[j]], sem_s[b]).wait()
          jn = jnp.minimum(j + NB, CH - 1)
          pltpu.async_copy(rows_hbm.at[sidx.at[jn]], bufs[b], sem_g[b])
        return carry

      lax.fori_loop(0, CH // NB, body, 0)
      for b in range(NB):
        pltpu.make_async_copy(rows_hbm.at[sidx.at[CH - 1]], bufs[b],
                              sem_g[b]).wait()
      plsc.subcore_barrier()
      pltpu.sync_copy(acc.at[pl.ds(s * SLAB, SLAB)],
                      out_hbm.at[p, c, pl.ds(s * SLAB, SLAB)])
      plsc.subcore_barrier()

  return agg


def _make_sc_degree(NP, CH):
  SLAB = NP // NS
  mesh = plsc.VectorSubcoreMesh(core_axis_name="c", subcore_axis_name="s")

  @functools.partial(
      pl.kernel,
      out_type=jax.ShapeDtypeStruct((NC, NP, 16), jnp.float32),
      mesh=mesh,
      compiler_params=pltpu.CompilerParams(use_tc_tiling_on_sc=False),
      scratch_types=[
          pltpu.VMEM((CH, K), jnp.int32),
          pltpu.VMEM((K, 16), jnp.float32),
          pltpu.VMEM_SHARED((NP, 16), jnp.float32),
      ],
  )
  def degk(dst_hbm, ones_hbm, z_hbm, out_hbm, didx, onesv, acc):
    c = lax.axis_index("c")
    s = lax.axis_index("s")
    w = s * NC + c
    pltpu.sync_copy(z_hbm, acc.at[pl.ds(s * SLAB, SLAB)])
    pltpu.sync_copy(ones_hbm, onesv)
    pltpu.sync_copy(dst_hbm.at[pl.ds(w * CH, CH)], didx)
    plsc.subcore_barrier()

    def body(j, carry):
      pltpu.sync_copy(onesv, acc.at[didx.at[j]], add=True)
      return carry

    lax.fori_loop(0, CH, body, 0)
    plsc.subcore_barrier()
    pltpu.sync_copy(acc.at[pl.ds(s * SLAB, SLAB)],
                    out_hbm.at[c, pl.ds(s * SLAB, SLAB)])

  return degk


def _dinv_of(deg_ref):
  deg = deg_ref[0, :, 0:1] + deg_ref[1, :, 0:1] + 1.0
  return lax.rsqrt(deg)


def _tc1_body(x_ref, w_ref, deg_ref, o_ref):
  dinv = _dinv_of(deg_ref)
  o_ref[...] = jnp.dot(x_ref[...], w_ref[...],
                       preferred_element_type=jnp.float32) * dinv


def _tc2_body(sp_ref, h_ref, deg_ref, b1_ref, w2_ref, o_ref):
  dinv = _dinv_of(deg_ref)
  s = jnp.concatenate(
      [sp_ref[0, 0] + sp_ref[0, 1], sp_ref[1, 0] + sp_ref[1, 1]], axis=1)
  pre = (s + h_ref[...]) * dinv + b1_ref[...]
  h = jnp.maximum(pre, 0.0)
  o_ref[...] = jnp.dot(h, w2_ref[...],
                       preferred_element_type=jnp.float32) * dinv


def _tc3_body(s2_ref, g_ref, deg_ref, x_ref, ws_ref, bv_ref, o_ref):
  dinv = _dinv_of(deg_ref)
  o_ref[...] = ((s2_ref[0, 0] + s2_ref[0, 1] + g_ref[...]) * dinv
                + jnp.dot(x_ref[...], ws_ref[...],
                          preferred_element_type=jnp.float32)
                + bv_ref[...])


def kernel(x, edge_index, W1, b1, W2, b2, Ws, bs):
  N, DIN = x.shape
  DH = W1.shape[1]
  DO = W2.shape[1]
  E = edge_index.shape[1]
  f32 = jnp.float32

  NP = -(-(N + 1) // 256) * 256
  SLAB = NP // NS
  CH = -(-E // (NC * NS * K))
  CH = -(-CH // 4) * 4
  EP = NC * NS * CH * K

  src = edge_index[0]
  dst = edge_index[1]
  epad = jnp.full((EP - E,), N, dtype=jnp.int32)
  srcp = jnp.concatenate([src, epad]).reshape(NC * NS * CH, K)
  dstp = jnp.concatenate([dst, epad]).reshape(NC * NS * CH, K)
  xp = jnp.pad(x, ((0, NP - N), (0, 0)))
  W2p = jnp.pad(W2, ((0, 0), (0, 16 - DO)))
  Wsp = jnp.pad(Ws, ((0, 0), (0, 16 - DO)))
  bv = jnp.pad((b2 + bs).reshape(1, DO), ((0, 0), (0, 16 - DO)))
  b1r = b1.reshape(1, DH)
  ones16 = jnp.ones((K, 16), f32)
  z_64 = jnp.zeros((SLAB, 64), f32)
  z_16 = jnp.zeros((SLAB, 16), f32)

  degp = _make_sc_degree(NP, CH)(dstp, ones16, z_16)

  BM = 1024
  grid = (NP // BM,)
  h1p = pl.pallas_call(
      _tc1_body,
      grid=grid,
      in_specs=[
          pl.BlockSpec((BM, DIN), lambda i: (i, 0)),
          pl.BlockSpec((DIN, DH), lambda i: (0, 0)),
          pl.BlockSpec((NC, BM, 16), lambda i: (0, i, 0)),
      ],
      out_specs=pl.BlockSpec((BM, DH), lambda i: (i, 0)),
      out_shape=jax.ShapeDtypeStruct((NP, DH), f32),
  )(xp, W1, degp)

  h1a = h1p[:, :64]
  h1b = h1p[:, 64:]
  Sp = _make_sc_agg(NP, 64, CH, 2)(h1a, h1b, srcp, dstp, z_64)

  gp = pl.pallas_call(
      _tc2_body,
      grid=grid,
      in_specs=[
          pl.BlockSpec((2, NC, BM, 64), lambda i: (0, 0, i, 0)),
          pl.BlockSpec((BM, DH), lambda i: (i, 0)),
          pl.BlockSpec((NC, BM, 16), lambda i: (0, i, 0)),
          pl.BlockSpec((1, DH), lambda i: (0, 0)),
          pl.BlockSpec((DH, 16), lambda i: (0, 0)),
      ],
      out_specs=pl.BlockSpec((BM, 16), lambda i: (i, 0)),
      out_shape=jax.ShapeDtypeStruct((NP, 16), f32),
  )(Sp, h1p, degp, b1r, W2p)

  S2p = _make_sc_agg(NP, 16, CH, 1)(gp, srcp, dstp, z_16)

  res = pl.pallas_call(
      _tc3_body,
      grid=grid,
      in_specs=[
          pl.BlockSpec((1, NC, BM, 16), lambda i: (0, 0, i, 0)),
          pl.BlockSpec((BM, 16), lambda i: (i, 0)),
          pl.BlockSpec((NC, BM, 16), lambda i: (0, i, 0)),
          pl.BlockSpec((BM, DIN), lambda i: (i, 0)),
          pl.BlockSpec((DIN, 16), lambda i: (0, 0)),
          pl.BlockSpec((1, 16), lambda i: (0, 0)),
      ],
      out_specs=pl.BlockSpec((BM, 16), lambda i: (i, 0)),
      out_shape=jax.ShapeDtypeStruct((NP, 16), f32),
  )(S2p, gp, degp, xp, Wsp, bv)

  return res[:N, :DO]


# speedup vs baseline: 1.6476x; 1.6476x over previous
import functools

import jax
import jax.numpy as jnp
from jax import lax
from jax.experimental import pallas as pl
from jax.experimental.pallas import tpu as pltpu
from jax.experimental.pallas import tpu_sc as plsc

NC = 2
NS = 16
K = 128


def _make_sc_agg(NP, W, CH, dtype):
  SLAB = NP // NS
  mesh = plsc.VectorSubcoreMesh(core_axis_name="c", subcore_axis_name="s")

  @functools.partial(
      pl.kernel,
      out_type=jax.ShapeDtypeStruct((NC, NP, W), dtype),
      mesh=mesh,
      compiler_params=pltpu.CompilerParams(use_tc_tiling_on_sc=False),
      scratch_types=[
          pltpu.VMEM((CH, K), jnp.int32),
          pltpu.VMEM((CH, K), jnp.int32),
          pltpu.VMEM((K, W), dtype),
          pltpu.VMEM((K, W), dtype),
          pltpu.VMEM_SHARED((NP, W), dtype),
          pltpu.SemaphoreType.DMA,
          pltpu.SemaphoreType.DMA,
      ],
  )
  def agg(rows_hbm, src_hbm, dst_hbm, z_hbm, out_hbm,
          sidx, didx, buf0, buf1, acc, sem0, sem1):
    c = lax.axis_index("c")
    s = lax.axis_index("s")
    w = s * NC + c
    pltpu.sync_copy(z_hbm, acc.at[pl.ds(s * SLAB, SLAB)])
    pltpu.sync_copy(src_hbm.at[pl.ds(w * CH, CH)], sidx)
    pltpu.sync_copy(dst_hbm.at[pl.ds(w * CH, CH)], didx)
    plsc.subcore_barrier()

    pltpu.async_copy(rows_hbm.at[sidx.at[0]], buf0, sem0)

    def body(i, carry):
      j0 = 2 * i
      j1 = 2 * i + 1
      pltpu.async_copy(rows_hbm.at[sidx.at[j1]], buf1, sem1)
      pltpu.make_async_copy(rows_hbm.at[sidx.at[j0]], buf0, sem0).wait()
      pltpu.sync_copy(buf0, acc.at[didx.at[j0]], add=True)
      jn = jnp.minimum(j0 + 2, CH - 1)
      pltpu.async_copy(rows_hbm.at[sidx.at[jn]], buf0, sem0)
      pltpu.make_async_copy(rows_hbm.at[sidx.at[j1]], buf1, sem1).wait()
      pltpu.sync_copy(buf1, acc.at[didx.at[j1]], add=True)
      return carry

    lax.fori_loop(0, CH // 2, body, 0)
    pltpu.make_async_copy(rows_hbm.at[sidx.at[CH - 1]], buf0, sem0).wait()
    plsc.subcore_barrier()
    pltpu.sync_copy(acc.at[pl.ds(s * SLAB, SLAB)],
                    out_hbm.at[c, pl.ds(s * SLAB, SLAB)])

  return agg


def _make_sc_degree(NP, CH):
  SLAB = NP // NS
  mesh = plsc.VectorSubcoreMesh(core_axis_name="c", subcore_axis_name="s")

  @functools.partial(
      pl.kernel,
      out_type=jax.ShapeDtypeStruct((NC, NP, 8), jnp.float32),
      mesh=mesh,
      compiler_params=pltpu.CompilerParams(use_tc_tiling_on_sc=False),
      scratch_types=[
          pltpu.VMEM((CH, K), jnp.int32),
          pltpu.VMEM((K, 8), jnp.float32),
          pltpu.VMEM_SHARED((NP, 8), jnp.float32),
      ],
  )
  def degk(dst_hbm, ones_hbm, z_hbm, out_hbm, didx, onesv, acc):
    c = lax.axis_index("c")
    s = lax.axis_index("s")
    w = s * NC + c
    pltpu.sync_copy(z_hbm, acc.at[pl.ds(s * SLAB, SLAB)])
    pltpu.sync_copy(ones_hbm, onesv)
    pltpu.sync_copy(dst_hbm.at[pl.ds(w * CH, CH)], didx)
    plsc.subcore_barrier()

    def body(j, carry):
      pltpu.sync_copy(onesv, acc.at[didx.at[j]], add=True)
      return carry

    lax.fori_loop(0, CH, body, 0)
    plsc.subcore_barrier()
    pltpu.sync_copy(acc.at[pl.ds(s * SLAB, SLAB)],
                    out_hbm.at[c, pl.ds(s * SLAB, SLAB)])

  return degk


def _dinv_of(deg_ref):
  deg = deg_ref[0, :, 0:1] + deg_ref[1, :, 0:1] + 1.0
  return lax.rsqrt(deg)


def _tc1_body(x_ref, w_ref, deg_ref, o_ref):
  dinv = _dinv_of(deg_ref)
  o_ref[...] = (jnp.dot(x_ref[...], w_ref[...],
                        preferred_element_type=jnp.float32)
                * dinv).astype(jnp.bfloat16)


def _tc2_body(sp_ref, h_ref, deg_ref, b1_ref, w2_ref, o_ref):
  dinv = _dinv_of(deg_ref)
  s = (sp_ref[0].astype(jnp.float32) + sp_ref[1].astype(jnp.float32)
       + h_ref[...].astype(jnp.float32))
  pre = s * dinv + b1_ref[...]
  h = jnp.maximum(pre, 0.0)
  o_ref[...] = jnp.dot(h, w2_ref[...],
                       preferred_element_type=jnp.float32) * dinv


def _tc3_body(s2_ref, g_ref, deg_ref, x_ref, ws_ref, bv_ref, o_ref):
  dinv = _dinv_of(deg_ref)
  o_ref[...] = ((s2_ref[0] + s2_ref[1] + g_ref[...]) * dinv
                + jnp.dot(x_ref[...], ws_ref[...],
                          preferred_element_type=jnp.float32)
                + bv_ref[...])


def kernel(x, edge_index, W1, b1, W2, b2, Ws, bs):
  N, DIN = x.shape
  DH = W1.shape[1]
  DO = W2.shape[1]
  E = edge_index.shape[1]
  f32 = jnp.float32

  NP = -(-(N + 1) // 256) * 256
  SLAB = NP // NS
  CH = -(-E // (NC * NS * K))
  CH += CH % 2
  EP = NC * NS * CH * K

  src = edge_index[0]
  dst = edge_index[1]
  epad = jnp.full((EP - E,), N, dtype=jnp.int32)
  srcp = jnp.concatenate([src, epad]).reshape(NC * NS * CH, K)
  dstp = jnp.concatenate([dst, epad]).reshape(NC * NS * CH, K)
  xp = jnp.pad(x, ((0, NP - N), (0, 0)))
  W2p = jnp.pad(W2, ((0, 0), (0, 8 - DO)))
  Wsp = jnp.pad(Ws, ((0, 0), (0, 8 - DO)))
  bv = jnp.pad((b2 + bs).reshape(1, DO), ((0, 0), (0, 8 - DO)))
  b1r = b1.reshape(1, DH)
  ones8 = jnp.ones((K, 8), f32)
  z_dh = jnp.zeros((SLAB, DH), jnp.bfloat16)
  z_8 = jnp.zeros((SLAB, 8), f32)

  degp = _make_sc_degree(NP, CH)(dstp, ones8, z_8)

  BM = 1024
  grid = (NP // BM,)
  h1p = pl.pallas_call(
      _tc1_body,
      grid=grid,
      in_specs=[
          pl.BlockSpec((BM, DIN), lambda i: (i, 0)),
          pl.BlockSpec((DIN, DH), lambda i: (0, 0)),
          pl.BlockSpec((NC, BM, 8), lambda i: (0, i, 0)),
      ],
      out_specs=pl.BlockSpec((BM, DH), lambda i: (i, 0)),
      out_shape=jax.ShapeDtypeStruct((NP, DH), jnp.bfloat16),
  )(xp, W1, degp)

  Sp = _make_sc_agg(NP, DH, CH, jnp.bfloat16)(h1p, srcp, dstp, z_dh)

  gp = pl.pallas_call(
      _tc2_body,
      grid=grid,
      in_specs=[
          pl.BlockSpec((NC, BM, DH), lambda i: (0, i, 0)),
          pl.BlockSpec((BM, DH), lambda i: (i, 0)),
          pl.BlockSpec((NC, BM, 8), lambda i: (0, i, 0)),
          pl.BlockSpec((1, DH), lambda i: (0, 0)),
          pl.BlockSpec((DH, 8), lambda i: (0, 0)),
      ],
      out_specs=pl.BlockSpec((BM, 8), lambda i: (i, 0)),
      out_shape=jax.ShapeDtypeStruct((NP, 8), f32),
  )(Sp, h1p, degp, b1r, W2p)

  S2p = _make_sc_agg(NP, 8, CH, f32)(gp, srcp, dstp, z_8)

  res = pl.pallas_call(
      _tc3_body,
      grid=grid,
      in_specs=[
          pl.BlockSpec((NC, BM, 8), lambda i: (0, i, 0)),
          pl.BlockSpec((BM, 8), lambda i: (i, 0)),
          pl.BlockSpec((NC, BM, 8), lambda i: (0, i, 0)),
          pl.BlockSpec((BM, DIN), lambda i: (i, 0)),
          pl.BlockSpec((DIN, 8), lambda i: (0, 0)),
          pl.BlockSpec((1, 8), lambda i: (0, 0)),
      ],
      out_specs=pl.BlockSpec((BM, 8), lambda i: (i, 0)),
      out_shape=jax.ShapeDtypeStruct((NP, 8), f32),
  )(S2p, gp, degp, xp, Wsp, bv)

  return res[:N, :DO]
